# baseline (speedup 1.0000x reference)
import functools

import jax
import jax.numpy as jnp
from jax import lax
from jax.experimental import pallas as pl
from jax.experimental.pallas import tpu as pltpu
from jax.experimental.pallas import tpu_sc as plsc

_NW = 32
_CH = 128

_INV2PI = 0.15915494309189535
_TWOPI = 6.283185307179586
_S1 = 9.9998459345e-01
_S3 = -1.6663259377e-01
_S5 = 8.3123882797e-03
_S7 = -1.9316269889e-04
_S9 = 2.1732569601e-06


def _psin(x):
    n = jnp.floor(x * _INV2PI + 0.5)
    r = x - n * _TWOPI
    r2 = r * r
    return r * (_S1 + r2 * (_S3 + r2 * (_S5 + r2 * (_S7 + r2 * _S9))))


def _sc_gather(table2, idx3):
    nw, n_chunks, ch = idx3.shape
    dw = table2.shape[1]
    rows = nw * n_chunks * ch
    mesh = plsc.VectorSubcoreMesh(core_axis_name="c", subcore_axis_name="s")

    @functools.partial(
        pl.kernel,
        mesh=mesh,
        out_type=jax.ShapeDtypeStruct((rows, dw), jnp.float32),
        compiler_params=pltpu.CompilerParams(use_tc_tiling_on_sc=True),
        scratch_types=[
            pltpu.VMEM((n_chunks, ch), jnp.int32),
            pltpu.VMEM((ch, dw), jnp.float32),
            pltpu.VMEM((ch, dw), jnp.float32),
            pltpu.SemaphoreType.DMA,
            pltpu.SemaphoreType.DMA,
            pltpu.SemaphoreType.DMA,
            pltpu.SemaphoreType.DMA,
        ],
    )
    def k(table_hbm, idx_hbm, out_hbm, idx_v, bufa, bufb, sga, sgb, swa, swb):
        wid = lax.axis_index("s") * 2 + lax.axis_index("c")
        base = wid * (n_chunks * ch)
        pltpu.sync_copy(idx_hbm.at[wid], idx_v)
        pltpu.async_copy(table_hbm.at[idx_v.at[0]], bufa, sga)
        pltpu.async_copy(table_hbm.at[idx_v.at[1]], bufb, sgb)

        def step(g, carry):
            j0 = 2 * g
            j1 = j0 + 1
            pltpu.make_async_copy(table_hbm.at[idx_v.at[0]], bufa, sga).wait()
            pltpu.async_copy(bufa, out_hbm.at[pl.ds(base + j0 * ch, ch)], swa)
            pltpu.make_async_copy(table_hbm.at[idx_v.at[0]], bufb, sgb).wait()
            pltpu.async_copy(bufb, out_hbm.at[pl.ds(base + j1 * ch, ch)], swb)

            @pl.when(j0 + 2 < n_chunks)
            def _():
                pltpu.make_async_copy(
                    bufa, out_hbm.at[pl.ds(base, ch)], swa).wait()
                pltpu.async_copy(table_hbm.at[idx_v.at[j0 + 2]], bufa, sga)

            @pl.when(j1 + 2 < n_chunks)
            def _():
                pltpu.make_async_copy(
                    bufb, out_hbm.at[pl.ds(base, ch)], swb).wait()
                pltpu.async_copy(table_hbm.at[idx_v.at[j1 + 2]], bufb, sgb)

            return carry

        lax.fori_loop(0, n_chunks // 2, step, 0)
        pltpu.make_async_copy(bufa, out_hbm.at[pl.ds(base, ch)], swa).wait()
        pltpu.make_async_copy(bufb, out_hbm.at[pl.ds(base, ch)], swb).wait()

    return k(table2, idx3)


def _tc_fuse(g2, par_col, t_col, v_col, nsf_col, nvf_col,
             w0, b0, t2wl, t2bl, tpw0, tpw1m, tpb, valw, valb, d):
    rows = g2.shape[0]
    blk = 2048
    grid = rows // blk

    def body(g_ref, par_ref, t_ref, v_ref, nsf_ref, nvf_ref,
             w0_ref, b0_ref, t2wl_ref, t2bl_ref, tpw0_ref, tpw1m_ref,
             tpb_ref, valw_ref, valb_ref, out_ref):
        g = g_ref[...]
        emb = jnp.where(par_ref[...] > 0, g[:, d:], g[:, :d])
        t = t_ref[...]
        lin = t * w0_ref[0, 0] + b0_ref[0, 0]
        s = _psin(t * t2wl_ref[...] + t2bl_ref[...])
        proj = (lin * tpw0_ref[...]
                + jnp.dot(s, tpw1m_ref[...],
                          preferred_element_type=jnp.float32)
                + tpb_ref[...])
        time_emb = proj * nsf_ref[...]
        val_emb = (v_ref[...] * valw_ref[...] + valb_ref[...]) * nvf_ref[...]
        out_ref[...] = emb + time_emb + val_emb

    full = lambda shape: pl.BlockSpec(shape, lambda i: (0, 0))
    row_blk = lambda w: pl.BlockSpec((blk, w), lambda i: (i, 0))
    return pl.pallas_call(
        body,
        grid=(grid,),
        in_specs=[
            row_blk(2 * d), row_blk(1), row_blk(1), row_blk(1), row_blk(1),
            row_blk(1),
            full((1, 1)), full((1, 1)), full(t2wl.shape), full(t2bl.shape),
            full(tpw0.shape), full(tpw1m.shape), full(tpb.shape),
            full(valw.shape), full(valb.shape),
        ],
        out_specs=row_blk(d),
        out_shape=jax.ShapeDtypeStruct((rows, d), jnp.float32),
    )(g2, par_col, t_col, v_col, nsf_col, nvf_col,
      w0, b0, t2wl, t2bl, tpw0, tpw1m, tpb, valw, valb)


def kernel(static_mask, code, numeric_value, time_delta_days,
           numeric_value_mask, table, t2v_w0, t2v_b0, t2v_W, t2v_B,
           tp_W, tp_b, val_W, val_b):
    b, s = code.shape
    d = table.shape[1]
    bs = b * s
    n_chunks = bs // (_NW * _CH)

    code_t = code.T.astype(jnp.int32)
    idx3 = (code_t >> 1).reshape(_NW, n_chunks, _CH)
    par_col = (code_t & 1).astype(jnp.float32).reshape(bs, 1)
    t_col = time_delta_days.T.reshape(bs, 1)
    v_col = numeric_value.T.reshape(bs, 1)
    nsf_col = (~static_mask).T.reshape(bs, 1).astype(jnp.float32)
    nvf_col = numeric_value_mask.T.reshape(bs, 1).astype(jnp.float32)

    table2 = table.reshape(bs_pairs := table.shape[0] // 2, 2 * d)
    g2 = _sc_gather(table2, idx3)

    k = t2v_W.shape[0]
    t2wl = jnp.zeros((1, d), jnp.float32).at[0, :k].set(t2v_W)
    t2bl = jnp.zeros((1, d), jnp.float32).at[0, :k].set(t2v_B)
    tpw1m = jnp.zeros((d, d), jnp.float32).at[:k, :].set(tp_W[1:, :])

    out = _tc_fuse(
        g2, par_col, t_col, v_col, nsf_col, nvf_col,
        t2v_w0.reshape(1, 1), t2v_b0.reshape(1, 1),
        t2wl, t2bl,
        tp_W[0:1, :], tpw1m, tp_b.reshape(1, -1),
        val_W.reshape(1, -1), val_b.reshape(1, -1), d)

    return out.reshape(s, b, d).transpose(1, 0, 2)
